# baseline (speedup 1.0000x reference)
import functools

import jax
import jax.numpy as jnp
from jax import lax
from jax.experimental import pallas as pl
from jax.experimental.pallas import tpu as pltpu
from jax.experimental.pallas import tpu_sc as plsc

N = 10000
E = 320000
IN_FEATS = 128
OUT_FEATS = 128
NEG_SLOPE = 0.2

NPAD = 10240
ROWS_PER_TILE = NPAD // 16
EDGE_BLOCK = 96
EG = EDGE_BLOCK // 16
CHUNK_BLOCKS = 6
NUM_CHUNKS = 35
BLOCKS_PER_TILE = NUM_CHUNKS * CHUNK_BLOCKS
EDGES_PER_TILE = BLOCKS_PER_TILE * EDGE_BLOCK
E_PAD = EDGES_PER_TILE * 16
HALF = 64
NORM_ROWS = 80


def _proj_kernel(feat_ref, w_ref, asrc_ref, adst_ref, xs_ref, alpha_s_ref,
                 alpha_d_ref):
    x = jnp.dot(feat_ref[...], w_ref[...], preferred_element_type=jnp.float32)
    xs_ref[0] = x[:, :HALF]
    xs_ref[1] = x[:, HALF:]
    alpha_s_ref[...] = jnp.sum(x * asrc_ref[...], axis=1)
    alpha_d_ref[...] = jnp.sum(x * adst_ref[...], axis=1)


def _project(feat_p, W, att_src, att_dst):
    blk = 256
    grid = (NPAD // blk,)
    return pl.pallas_call(
        _proj_kernel,
        grid=grid,
        in_specs=[
            pl.BlockSpec((blk, IN_FEATS), lambda i: (i, 0)),
            pl.BlockSpec((IN_FEATS, OUT_FEATS), lambda i: (0, 0)),
            pl.BlockSpec((1, OUT_FEATS), lambda i: (0, 0)),
            pl.BlockSpec((1, OUT_FEATS), lambda i: (0, 0)),
        ],
        out_specs=[
            pl.BlockSpec((2, blk, HALF), lambda i: (0, i, 0)),
            pl.BlockSpec((blk,), lambda i: (i,)),
            pl.BlockSpec((blk,), lambda i: (i,)),
        ],
        out_shape=[
            jax.ShapeDtypeStruct((2, NPAD, HALF), jnp.float32),
            jax.ShapeDtypeStruct((NPAD,), jnp.float32),
            jax.ShapeDtypeStruct((NPAD,), jnp.float32),
        ],
    )(feat_p, W, att_src, att_dst)


def _edge_kernel(x_hbm, asrc_hbm, adst_hbm, edges_hbm, z2d_hbm,
                 zdn_hbm, out_hbm,
                 x_sh, acc_sh, den_sh,
                 asrc_v, adst_v, sd_v, rows_a, rows_b, rows_c,
                 wrow_a, wrow_b, wrow_c, dbuf_v,
                 sem_g0, sem_g1, sem_g2, sem_s0, sem_s1, sem_s2,
                 sem_d0, sem_d1, sem_d2):
    c = lax.axis_index("c")
    s = lax.axis_index("s")
    row0 = s * ROWS_PER_TILE
    iota = lax.iota(jnp.int32, 16)
    zeros_i = jnp.zeros((16,), jnp.int32)

    pltpu.sync_copy(x_hbm.at[c, pl.ds(row0, ROWS_PER_TILE)],
                    x_sh.at[pl.ds(row0, ROWS_PER_TILE)])
    pltpu.sync_copy(z2d_hbm.at[pl.ds(row0, ROWS_PER_TILE)],
                    acc_sh.at[pl.ds(row0, ROWS_PER_TILE)])
    pltpu.sync_copy(zdn_hbm.at[pl.ds(row0, ROWS_PER_TILE)],
                    den_sh.at[pl.ds(row0, ROWS_PER_TILE)])
    pltpu.sync_copy(zdn_hbm.at[pl.ds(0, EDGE_BLOCK)], wrow_a)
    pltpu.sync_copy(zdn_hbm.at[pl.ds(0, EDGE_BLOCK)], wrow_b)
    pltpu.sync_copy(zdn_hbm.at[pl.ds(0, EDGE_BLOCK)], wrow_c)
    pltpu.sync_copy(asrc_hbm, asrc_v)
    pltpu.sync_copy(adst_hbm, adst_v)
    plsc.subcore_barrier()

    def _weights(b, wrow):
        @pl.loop(0, EG)
        def _w(j):
            srcv = sd_v[0, b, pl.ds(j * 16, 16)]
            dstv = sd_v[1, b, pl.ds(j * 16, 16)]
            e = (plsc.load_gather(asrc_v, [srcv])
                 + plsc.load_gather(adst_v, [dstv]))
            e = jnp.where(e > 0, e, jnp.float32(NEG_SLOPE) * e)
            w = jnp.exp(e)
            plsc.store_scatter(wrow, [j * 16 + iota, zeros_i], w)

    def _scale(rows, wrow):
        @pl.loop(0, EG)
        def _s(g):
            w16 = plsc.load_gather(wrow, [g * 16 + iota, zeros_i])
            for kk in range(16):
                w = w16[kk]
                k = g * 16 + kk
                for j in range(4):
                    rows[k, pl.ds(j * 16, 16)] = (
                        rows[k, pl.ds(j * 16, 16)] * w)

    rows = (rows_a, rows_b, rows_c)
    wrows = (wrow_a, wrow_b, wrow_c)
    gsems = (sem_g0, sem_g1, sem_g2)
    ssems = (sem_s0, sem_s1, sem_s2)
    dsems = (sem_d0, sem_d1, sem_d2)

    @pl.loop(0, NUM_CHUNKS)
    def _chunk(ch):
        pltpu.sync_copy(edges_hbm.at[s, ch], sd_v)

        g = [None] * CHUNK_BLOCKS
        sc = [None] * CHUNK_BLOCKS
        dn = [None] * CHUNK_BLOCKS
        g[0] = pltpu.async_copy(x_sh.at[sd_v.at[0, 0]], rows[0], gsems[0])
        _weights(0, wrows[0])
        for i in range(CHUNK_BLOCKS):
            k = i % 3
            g[i].wait()
            if i >= 2:
                sc[i - 2].wait()
                dn[i - 2].wait()
            if i + 1 < CHUNK_BLOCKS:
                g[i + 1] = pltpu.async_copy(x_sh.at[sd_v.at[0, i + 1]],
                                            rows[(i + 1) % 3],
                                            gsems[(i + 1) % 3])
            _scale(rows[k], wrows[k])
            sc[i] = pltpu.async_copy(rows[k], acc_sh.at[sd_v.at[1, i]],
                                     ssems[k], add=True)
            dn[i] = pltpu.async_copy(wrows[k], den_sh.at[sd_v.at[1, i]],
                                     dsems[k], add=True)
            if i + 1 < CHUNK_BLOCKS:
                _weights(i + 1, wrows[(i + 1) % 3])
        sc[CHUNK_BLOCKS - 2].wait()
        dn[CHUNK_BLOCKS - 2].wait()
        sc[CHUNK_BLOCKS - 1].wait()
        dn[CHUNK_BLOCKS - 1].wait()

    plsc.subcore_barrier()

    @pl.loop(0, ROWS_PER_TILE // NORM_ROWS)
    def _norm_chunk(cb):
        base = row0 + cb * NORM_ROWS
        pltpu.sync_copy(den_sh.at[pl.ds(base, NORM_ROWS)],
                        wrow_a.at[pl.ds(0, NORM_ROWS)])

        @pl.loop(0, NORM_ROWS // 16)
        def _inv(g):
            d = plsc.load_gather(wrow_a, [g * 16 + iota, zeros_i])
            dbuf_v[pl.ds(g * 16, 16)] = (jnp.float32(1.0)
                                         / (d + jnp.float32(1e-16)))

        pltpu.sync_copy(acc_sh.at[pl.ds(base, NORM_ROWS)],
                        rows_a.at[pl.ds(0, NORM_ROWS)])

        @pl.loop(0, NORM_ROWS // 16)
        def _norm(g):
            m16 = dbuf_v[pl.ds(g * 16, 16)]
            for kk in range(16):
                m = m16[kk]
                k = g * 16 + kk
                for j in range(4):
                    rows_a[k, pl.ds(j * 16, 16)] = (
                        rows_a[k, pl.ds(j * 16, 16)] * m)

        pltpu.sync_copy(rows_a.at[pl.ds(0, NORM_ROWS)],
                        out_hbm.at[c, pl.ds(base, NORM_ROWS)])


def _edge_phase(x_split, alpha_s, alpha_d, edges):
    z2d = jnp.zeros((NPAD, HALF), jnp.float32)
    zdn = jnp.zeros((NPAD, 8), jnp.float32)
    mesh = plsc.VectorSubcoreMesh(core_axis_name="c", subcore_axis_name="s")
    f = pl.kernel(
        _edge_kernel,
        out_type=jax.ShapeDtypeStruct((2, NPAD, HALF), jnp.float32),
        mesh=mesh,
        compiler_params=pltpu.CompilerParams(needs_layout_passes=False,
                                             use_tc_tiling_on_sc=False),
        scratch_types=[
            pltpu.VMEM_SHARED((NPAD, HALF), jnp.float32),
            pltpu.VMEM_SHARED((NPAD, HALF), jnp.float32),
            pltpu.VMEM_SHARED((NPAD, 8), jnp.float32),
            pltpu.VMEM((NPAD,), jnp.float32),
            pltpu.VMEM((NPAD,), jnp.float32),
            pltpu.VMEM((2, CHUNK_BLOCKS, EDGE_BLOCK), jnp.int32),
            pltpu.VMEM((EDGE_BLOCK, HALF), jnp.float32),
            pltpu.VMEM((EDGE_BLOCK, HALF), jnp.float32),
            pltpu.VMEM((EDGE_BLOCK, HALF), jnp.float32),
            pltpu.VMEM((EDGE_BLOCK, 8), jnp.float32),
            pltpu.VMEM((EDGE_BLOCK, 8), jnp.float32),
            pltpu.VMEM((EDGE_BLOCK, 8), jnp.float32),
            pltpu.VMEM((EDGE_BLOCK,), jnp.float32),
            pltpu.SemaphoreType.DMA,
            pltpu.SemaphoreType.DMA,
            pltpu.SemaphoreType.DMA,
            pltpu.SemaphoreType.DMA,
            pltpu.SemaphoreType.DMA,
            pltpu.SemaphoreType.DMA,
            pltpu.SemaphoreType.DMA,
            pltpu.SemaphoreType.DMA,
            pltpu.SemaphoreType.DMA,
        ],
    )
    return f(x_split, alpha_s, alpha_d, edges, z2d, zdn)


def kernel(feat, edge_index, W, att_src, att_dst):
    feat_p = jnp.pad(feat, ((0, NPAD - N), (0, 0)))
    x_split, alpha_s, alpha_d = _project(feat_p, W, att_src, att_dst)

    src = edge_index[0].astype(jnp.int32)
    dst = edge_index[1].astype(jnp.int32)
    src = jnp.pad(src, (0, E_PAD - E))
    dst = jnp.pad(dst, (0, E_PAD - E), constant_values=NPAD - 1)
    edges = jnp.stack(
        [src.reshape(16, NUM_CHUNKS, CHUNK_BLOCKS, EDGE_BLOCK),
         dst.reshape(16, NUM_CHUNKS, CHUNK_BLOCKS, EDGE_BLOCK)], axis=2)

    out = _edge_phase(x_split, alpha_s, alpha_d, edges)
    out = jnp.concatenate([out[0, :N, :], out[1, :N, :]], axis=-1)
    return out.reshape(N, 1, OUT_FEATS)


# speedup vs baseline: 1.0141x; 1.0141x over previous
import functools

import jax
import jax.numpy as jnp
from jax import lax
from jax.experimental import pallas as pl
from jax.experimental.pallas import tpu as pltpu
from jax.experimental.pallas import tpu_sc as plsc

N = 10000
E = 320000
IN_FEATS = 128
OUT_FEATS = 128
NEG_SLOPE = 0.2

NPAD = 10240
ROWS_PER_TILE = NPAD // 16
EDGE_BLOCK = 96
EG = EDGE_BLOCK // 16
CHUNK_BLOCKS = 10
NUM_CHUNKS = 21
BLOCKS_PER_TILE = NUM_CHUNKS * CHUNK_BLOCKS
EDGES_PER_TILE = BLOCKS_PER_TILE * EDGE_BLOCK
E_PAD = EDGES_PER_TILE * 16
HALF = 64
NORM_ROWS = 80


def _proj_kernel(feat_ref, w_ref, asrc_ref, adst_ref, xs_ref, alpha_s_ref,
                 alpha_d_ref):
    x = jnp.dot(feat_ref[...], w_ref[...], preferred_element_type=jnp.float32)
    xs_ref[0] = x[:, :HALF]
    xs_ref[1] = x[:, HALF:]
    alpha_s_ref[...] = jnp.sum(x * asrc_ref[...], axis=1)
    alpha_d_ref[...] = jnp.sum(x * adst_ref[...], axis=1)


def _project(feat_p, W, att_src, att_dst):
    blk = 256
    grid = (NPAD // blk,)
    return pl.pallas_call(
        _proj_kernel,
        grid=grid,
        in_specs=[
            pl.BlockSpec((blk, IN_FEATS), lambda i: (i, 0)),
            pl.BlockSpec((IN_FEATS, OUT_FEATS), lambda i: (0, 0)),
            pl.BlockSpec((1, OUT_FEATS), lambda i: (0, 0)),
            pl.BlockSpec((1, OUT_FEATS), lambda i: (0, 0)),
        ],
        out_specs=[
            pl.BlockSpec((2, blk, HALF), lambda i: (0, i, 0)),
            pl.BlockSpec((blk,), lambda i: (i,)),
            pl.BlockSpec((blk,), lambda i: (i,)),
        ],
        out_shape=[
            jax.ShapeDtypeStruct((2, NPAD, HALF), jnp.float32),
            jax.ShapeDtypeStruct((NPAD,), jnp.float32),
            jax.ShapeDtypeStruct((NPAD,), jnp.float32),
        ],
    )(feat_p, W, att_src, att_dst)


def _edge_kernel(x_hbm, asrc_hbm, adst_hbm, edges_hbm, z2d_hbm,
                 zdn_hbm, out_hbm,
                 x_sh, acc_sh, den_sh,
                 asrc_v, adst_v, sd_v, rows_a, rows_b, rows_c,
                 wrow_a, wrow_b, wrow_c, dbuf_v,
                 sem_g0, sem_g1, sem_g2, sem_s0, sem_s1, sem_s2,
                 sem_d0, sem_d1, sem_d2):
    c = lax.axis_index("c")
    s = lax.axis_index("s")
    row0 = s * ROWS_PER_TILE
    iota = lax.iota(jnp.int32, 16)
    zeros_i = jnp.zeros((16,), jnp.int32)

    pltpu.sync_copy(x_hbm.at[c, pl.ds(row0, ROWS_PER_TILE)],
                    x_sh.at[pl.ds(row0, ROWS_PER_TILE)])
    pltpu.sync_copy(z2d_hbm.at[pl.ds(row0, ROWS_PER_TILE)],
                    acc_sh.at[pl.ds(row0, ROWS_PER_TILE)])
    pltpu.sync_copy(zdn_hbm.at[pl.ds(row0, ROWS_PER_TILE)],
                    den_sh.at[pl.ds(row0, ROWS_PER_TILE)])
    pltpu.sync_copy(zdn_hbm.at[pl.ds(0, EDGE_BLOCK)], wrow_a)
    pltpu.sync_copy(zdn_hbm.at[pl.ds(0, EDGE_BLOCK)], wrow_b)
    pltpu.sync_copy(zdn_hbm.at[pl.ds(0, EDGE_BLOCK)], wrow_c)
    pltpu.sync_copy(asrc_hbm, asrc_v)
    pltpu.sync_copy(adst_hbm, adst_v)
    plsc.subcore_barrier()

    def _weights(b, wrow):
        @pl.loop(0, EG)
        def _w(j):
            srcv = sd_v[0, b, pl.ds(j * 16, 16)]
            dstv = sd_v[1, b, pl.ds(j * 16, 16)]
            e = (plsc.load_gather(asrc_v, [srcv])
                 + plsc.load_gather(adst_v, [dstv]))
            e = jnp.where(e > 0, e, jnp.float32(NEG_SLOPE) * e)
            w = jnp.exp(e)
            plsc.store_scatter(wrow, [j * 16 + iota, zeros_i], w)

    def _scale(rows, wrow):
        @pl.loop(0, EG)
        def _s(g):
            w16 = plsc.load_gather(wrow, [g * 16 + iota, zeros_i])
            for kk in range(16):
                w = w16[kk]
                k = g * 16 + kk
                for j in range(4):
                    rows[k, pl.ds(j * 16, 16)] = (
                        rows[k, pl.ds(j * 16, 16)] * w)

    rows = (rows_a, rows_b, rows_c)
    wrows = (wrow_a, wrow_b, wrow_c)
    gsems = (sem_g0, sem_g1, sem_g2)
    ssems = (sem_s0, sem_s1, sem_s2)
    dsems = (sem_d0, sem_d1, sem_d2)

    @pl.loop(0, NUM_CHUNKS)
    def _chunk(ch):
        pltpu.sync_copy(edges_hbm.at[s, ch], sd_v)

        g = [None] * CHUNK_BLOCKS
        sc = [None] * CHUNK_BLOCKS
        dn = [None] * CHUNK_BLOCKS
        g[0] = pltpu.async_copy(x_sh.at[sd_v.at[0, 0]], rows[0], gsems[0])
        _weights(0, wrows[0])
        for i in range(CHUNK_BLOCKS):
            k = i % 3
            g[i].wait()
            if i >= 2:
                sc[i - 2].wait()
                dn[i - 2].wait()
            if i + 1 < CHUNK_BLOCKS:
                g[i + 1] = pltpu.async_copy(x_sh.at[sd_v.at[0, i + 1]],
                                            rows[(i + 1) % 3],
                                            gsems[(i + 1) % 3])
            _scale(rows[k], wrows[k])
            sc[i] = pltpu.async_copy(rows[k], acc_sh.at[sd_v.at[1, i]],
                                     ssems[k], add=True)
            dn[i] = pltpu.async_copy(wrows[k], den_sh.at[sd_v.at[1, i]],
                                     dsems[k], add=True)
            if i + 1 < CHUNK_BLOCKS:
                _weights(i + 1, wrows[(i + 1) % 3])
        sc[CHUNK_BLOCKS - 2].wait()
        dn[CHUNK_BLOCKS - 2].wait()
        sc[CHUNK_BLOCKS - 1].wait()
        dn[CHUNK_BLOCKS - 1].wait()

    plsc.subcore_barrier()

    @pl.loop(0, ROWS_PER_TILE // NORM_ROWS)
    def _norm_chunk(cb):
        base = row0 + cb * NORM_ROWS
        pltpu.sync_copy(den_sh.at[pl.ds(base, NORM_ROWS)],
                        wrow_a.at[pl.ds(0, NORM_ROWS)])

        @pl.loop(0, NORM_ROWS // 16)
        def _inv(g):
            d = plsc.load_gather(wrow_a, [g * 16 + iota, zeros_i])
            dbuf_v[pl.ds(g * 16, 16)] = (jnp.float32(1.0)
                                         / (d + jnp.float32(1e-16)))

        pltpu.sync_copy(acc_sh.at[pl.ds(base, NORM_ROWS)],
                        rows_a.at[pl.ds(0, NORM_ROWS)])

        @pl.loop(0, NORM_ROWS // 16)
        def _norm(g):
            m16 = dbuf_v[pl.ds(g * 16, 16)]
            for kk in range(16):
                m = m16[kk]
                k = g * 16 + kk
                for j in range(4):
                    rows_a[k, pl.ds(j * 16, 16)] = (
                        rows_a[k, pl.ds(j * 16, 16)] * m)

        pltpu.sync_copy(rows_a.at[pl.ds(0, NORM_ROWS)],
                        out_hbm.at[c, pl.ds(base, NORM_ROWS)])


def _edge_phase(x_split, alpha_s, alpha_d, edges):
    z2d = jnp.zeros((NPAD, HALF), jnp.float32)
    zdn = jnp.zeros((NPAD, 8), jnp.float32)
    mesh = plsc.VectorSubcoreMesh(core_axis_name="c", subcore_axis_name="s")
    f = pl.kernel(
        _edge_kernel,
        out_type=jax.ShapeDtypeStruct((2, NPAD, HALF), jnp.float32),
        mesh=mesh,
        compiler_params=pltpu.CompilerParams(needs_layout_passes=False,
                                             use_tc_tiling_on_sc=False),
        scratch_types=[
            pltpu.VMEM_SHARED((NPAD, HALF), jnp.float32),
            pltpu.VMEM_SHARED((NPAD, HALF), jnp.float32),
            pltpu.VMEM_SHARED((NPAD, 8), jnp.float32),
            pltpu.VMEM((NPAD,), jnp.float32),
            pltpu.VMEM((NPAD,), jnp.float32),
            pltpu.VMEM((2, CHUNK_BLOCKS, EDGE_BLOCK), jnp.int32),
            pltpu.VMEM((EDGE_BLOCK, HALF), jnp.float32),
            pltpu.VMEM((EDGE_BLOCK, HALF), jnp.float32),
            pltpu.VMEM((EDGE_BLOCK, HALF), jnp.float32),
            pltpu.VMEM((EDGE_BLOCK, 8), jnp.float32),
            pltpu.VMEM((EDGE_BLOCK, 8), jnp.float32),
            pltpu.VMEM((EDGE_BLOCK, 8), jnp.float32),
            pltpu.VMEM((EDGE_BLOCK,), jnp.float32),
            pltpu.SemaphoreType.DMA,
            pltpu.SemaphoreType.DMA,
            pltpu.SemaphoreType.DMA,
            pltpu.SemaphoreType.DMA,
            pltpu.SemaphoreType.DMA,
            pltpu.SemaphoreType.DMA,
            pltpu.SemaphoreType.DMA,
            pltpu.SemaphoreType.DMA,
            pltpu.SemaphoreType.DMA,
        ],
    )
    return f(x_split, alpha_s, alpha_d, edges, z2d, zdn)


def kernel(feat, edge_index, W, att_src, att_dst):
    feat_p = jnp.pad(feat, ((0, NPAD - N), (0, 0)))
    x_split, alpha_s, alpha_d = _project(feat_p, W, att_src, att_dst)

    src = edge_index[0].astype(jnp.int32)
    dst = edge_index[1].astype(jnp.int32)
    src = jnp.pad(src, (0, E_PAD - E))
    dst = jnp.pad(dst, (0, E_PAD - E), constant_values=NPAD - 1)
    edges = jnp.stack(
        [src.reshape(16, NUM_CHUNKS, CHUNK_BLOCKS, EDGE_BLOCK),
         dst.reshape(16, NUM_CHUNKS, CHUNK_BLOCKS, EDGE_BLOCK)], axis=2)

    out = _edge_phase(x_split, alpha_s, alpha_d, edges)
    out = jnp.concatenate([out[0, :N, :], out[1, :N, :]], axis=-1)
    return out.reshape(N, 1, OUT_FEATS)


# speedup vs baseline: 1.0321x; 1.0177x over previous
import functools

import jax
import jax.numpy as jnp
from jax import lax
from jax.experimental import pallas as pl
from jax.experimental.pallas import tpu as pltpu
from jax.experimental.pallas import tpu_sc as plsc

N = 10000
E = 320000
IN_FEATS = 128
OUT_FEATS = 128
NEG_SLOPE = 0.2

NPAD = 10240
ROWS_PER_TILE = NPAD // 16
EDGE_BLOCK = 96
EG = EDGE_BLOCK // 16
CHUNK_BLOCKS = 14
NUM_CHUNKS = 15
BLOCKS_PER_TILE = NUM_CHUNKS * CHUNK_BLOCKS
EDGES_PER_TILE = BLOCKS_PER_TILE * EDGE_BLOCK
E_PAD = EDGES_PER_TILE * 16
HALF = 64
NORM_ROWS = 80


def _proj_kernel(feat_ref, w_ref, asrc_ref, adst_ref, xs_ref, alpha_s_ref,
                 alpha_d_ref):
    x = jnp.dot(feat_ref[...], w_ref[...], preferred_element_type=jnp.float32)
    xs_ref[0] = x[:, :HALF]
    xs_ref[1] = x[:, HALF:]
    alpha_s_ref[...] = jnp.sum(x * asrc_ref[...], axis=1)
    alpha_d_ref[...] = jnp.sum(x * adst_ref[...], axis=1)


def _project(feat_p, W, att_src, att_dst):
    blk = 256
    grid = (NPAD // blk,)
    return pl.pallas_call(
        _proj_kernel,
        grid=grid,
        in_specs=[
            pl.BlockSpec((blk, IN_FEATS), lambda i: (i, 0)),
            pl.BlockSpec((IN_FEATS, OUT_FEATS), lambda i: (0, 0)),
            pl.BlockSpec((1, OUT_FEATS), lambda i: (0, 0)),
            pl.BlockSpec((1, OUT_FEATS), lambda i: (0, 0)),
        ],
        out_specs=[
            pl.BlockSpec((2, blk, HALF), lambda i: (0, i, 0)),
            pl.BlockSpec((blk,), lambda i: (i,)),
            pl.BlockSpec((blk,), lambda i: (i,)),
        ],
        out_shape=[
            jax.ShapeDtypeStruct((2, NPAD, HALF), jnp.float32),
            jax.ShapeDtypeStruct((NPAD,), jnp.float32),
            jax.ShapeDtypeStruct((NPAD,), jnp.float32),
        ],
    )(feat_p, W, att_src, att_dst)


def _edge_kernel(x_hbm, asrc_hbm, adst_hbm, edges_hbm, z2d_hbm,
                 zdn_hbm, out_hbm,
                 x_sh, acc_sh, den_sh,
                 asrc_v, adst_v, sd_v, rows_a, rows_b, rows_c,
                 wrow_a, wrow_b, wrow_c, dbuf_v,
                 sem_g0, sem_g1, sem_g2, sem_s0, sem_s1, sem_s2,
                 sem_d0, sem_d1, sem_d2):
    c = lax.axis_index("c")
    s = lax.axis_index("s")
    row0 = s * ROWS_PER_TILE
    iota = lax.iota(jnp.int32, 16)
    zeros_i = jnp.zeros((16,), jnp.int32)

    pltpu.sync_copy(x_hbm.at[c, pl.ds(row0, ROWS_PER_TILE)],
                    x_sh.at[pl.ds(row0, ROWS_PER_TILE)])
    pltpu.sync_copy(z2d_hbm.at[pl.ds(row0, ROWS_PER_TILE)],
                    acc_sh.at[pl.ds(row0, ROWS_PER_TILE)])
    pltpu.sync_copy(zdn_hbm.at[pl.ds(row0, ROWS_PER_TILE)],
                    den_sh.at[pl.ds(row0, ROWS_PER_TILE)])
    pltpu.sync_copy(zdn_hbm.at[pl.ds(0, EDGE_BLOCK)], wrow_a)
    pltpu.sync_copy(zdn_hbm.at[pl.ds(0, EDGE_BLOCK)], wrow_b)
    pltpu.sync_copy(zdn_hbm.at[pl.ds(0, EDGE_BLOCK)], wrow_c)
    pltpu.sync_copy(asrc_hbm, asrc_v)
    pltpu.sync_copy(adst_hbm, adst_v)
    plsc.subcore_barrier()

    def _weights(b, wrow):
        @pl.loop(0, EG)
        def _w(j):
            srcv = sd_v[0, b, pl.ds(j * 16, 16)]
            dstv = sd_v[1, b, pl.ds(j * 16, 16)]
            e = (plsc.load_gather(asrc_v, [srcv])
                 + plsc.load_gather(adst_v, [dstv]))
            e = jnp.where(e > 0, e, jnp.float32(NEG_SLOPE) * e)
            w = jnp.exp(e)
            plsc.store_scatter(wrow, [j * 16 + iota, zeros_i], w)

    def _scale(rows, wrow):
        @pl.loop(0, EG)
        def _s(g):
            w16 = plsc.load_gather(wrow, [g * 16 + iota, zeros_i])
            for kk in range(16):
                w = w16[kk]
                k = g * 16 + kk
                for j in range(4):
                    rows[k, pl.ds(j * 16, 16)] = (
                        rows[k, pl.ds(j * 16, 16)] * w)

    rows = (rows_a, rows_b, rows_c)
    wrows = (wrow_a, wrow_b, wrow_c)
    gsems = (sem_g0, sem_g1, sem_g2)
    ssems = (sem_s0, sem_s1, sem_s2)
    dsems = (sem_d0, sem_d1, sem_d2)

    @pl.loop(0, NUM_CHUNKS)
    def _chunk(ch):
        pltpu.sync_copy(edges_hbm.at[s, ch], sd_v)

        g = [None] * CHUNK_BLOCKS
        sc = [None] * CHUNK_BLOCKS
        dn = [None] * CHUNK_BLOCKS
        g[0] = pltpu.async_copy(x_sh.at[sd_v.at[0, 0]], rows[0], gsems[0])
        _weights(0, wrows[0])
        for i in range(CHUNK_BLOCKS):
            k = i % 3
            g[i].wait()
            if i >= 2:
                sc[i - 2].wait()
                dn[i - 2].wait()
            if i + 1 < CHUNK_BLOCKS:
                g[i + 1] = pltpu.async_copy(x_sh.at[sd_v.at[0, i + 1]],
                                            rows[(i + 1) % 3],
                                            gsems[(i + 1) % 3])
            _scale(rows[k], wrows[k])
            sc[i] = pltpu.async_copy(rows[k], acc_sh.at[sd_v.at[1, i]],
                                     ssems[k], add=True)
            dn[i] = pltpu.async_copy(wrows[k], den_sh.at[sd_v.at[1, i]],
                                     dsems[k], add=True)
            if i + 1 < CHUNK_BLOCKS:
                _weights(i + 1, wrows[(i + 1) % 3])
        sc[CHUNK_BLOCKS - 2].wait()
        dn[CHUNK_BLOCKS - 2].wait()
        sc[CHUNK_BLOCKS - 1].wait()
        dn[CHUNK_BLOCKS - 1].wait()

    plsc.subcore_barrier()

    @pl.loop(0, ROWS_PER_TILE // NORM_ROWS)
    def _norm_chunk(cb):
        base = row0 + cb * NORM_ROWS
        pltpu.sync_copy(den_sh.at[pl.ds(base, NORM_ROWS)],
                        wrow_a.at[pl.ds(0, NORM_ROWS)])

        @pl.loop(0, NORM_ROWS // 16)
        def _inv(g):
            d = plsc.load_gather(wrow_a, [g * 16 + iota, zeros_i])
            dbuf_v[pl.ds(g * 16, 16)] = (jnp.float32(1.0)
                                         / (d + jnp.float32(1e-16)))

        pltpu.sync_copy(acc_sh.at[pl.ds(base, NORM_ROWS)],
                        rows_a.at[pl.ds(0, NORM_ROWS)])

        @pl.loop(0, NORM_ROWS // 16)
        def _norm(g):
            m16 = dbuf_v[pl.ds(g * 16, 16)]
            for kk in range(16):
                m = m16[kk]
                k = g * 16 + kk
                for j in range(4):
                    rows_a[k, pl.ds(j * 16, 16)] = (
                        rows_a[k, pl.ds(j * 16, 16)] * m)

        pltpu.sync_copy(rows_a.at[pl.ds(0, NORM_ROWS)],
                        out_hbm.at[c, pl.ds(base, NORM_ROWS)])


def _edge_phase(x_split, alpha_s, alpha_d, edges):
    z2d = jnp.zeros((NPAD, HALF), jnp.float32)
    zdn = jnp.zeros((NPAD, 8), jnp.float32)
    mesh = plsc.VectorSubcoreMesh(core_axis_name="c", subcore_axis_name="s")
    f = pl.kernel(
        _edge_kernel,
        out_type=jax.ShapeDtypeStruct((2, NPAD, HALF), jnp.float32),
        mesh=mesh,
        compiler_params=pltpu.CompilerParams(needs_layout_passes=False,
                                             use_tc_tiling_on_sc=False),
        scratch_types=[
            pltpu.VMEM_SHARED((NPAD, HALF), jnp.float32),
            pltpu.VMEM_SHARED((NPAD, HALF), jnp.float32),
            pltpu.VMEM_SHARED((NPAD, 8), jnp.float32),
            pltpu.VMEM((NPAD,), jnp.float32),
            pltpu.VMEM((NPAD,), jnp.float32),
            pltpu.VMEM((2, CHUNK_BLOCKS, EDGE_BLOCK), jnp.int32),
            pltpu.VMEM((EDGE_BLOCK, HALF), jnp.float32),
            pltpu.VMEM((EDGE_BLOCK, HALF), jnp.float32),
            pltpu.VMEM((EDGE_BLOCK, HALF), jnp.float32),
            pltpu.VMEM((EDGE_BLOCK, 8), jnp.float32),
            pltpu.VMEM((EDGE_BLOCK, 8), jnp.float32),
            pltpu.VMEM((EDGE_BLOCK, 8), jnp.float32),
            pltpu.VMEM((EDGE_BLOCK,), jnp.float32),
            pltpu.SemaphoreType.DMA,
            pltpu.SemaphoreType.DMA,
            pltpu.SemaphoreType.DMA,
            pltpu.SemaphoreType.DMA,
            pltpu.SemaphoreType.DMA,
            pltpu.SemaphoreType.DMA,
            pltpu.SemaphoreType.DMA,
            pltpu.SemaphoreType.DMA,
            pltpu.SemaphoreType.DMA,
        ],
    )
    return f(x_split, alpha_s, alpha_d, edges, z2d, zdn)


def kernel(feat, edge_index, W, att_src, att_dst):
    feat_p = jnp.pad(feat, ((0, NPAD - N), (0, 0)))
    x_split, alpha_s, alpha_d = _project(feat_p, W, att_src, att_dst)

    src = edge_index[0].astype(jnp.int32)
    dst = edge_index[1].astype(jnp.int32)
    src = jnp.pad(src, (0, E_PAD - E))
    dst = jnp.pad(dst, (0, E_PAD - E), constant_values=NPAD - 1)
    edges = jnp.stack(
        [src.reshape(16, NUM_CHUNKS, CHUNK_BLOCKS, EDGE_BLOCK),
         dst.reshape(16, NUM_CHUNKS, CHUNK_BLOCKS, EDGE_BLOCK)], axis=2)

    out = _edge_phase(x_split, alpha_s, alpha_d, edges)
    out = jnp.concatenate([out[0, :N, :], out[1, :N, :]], axis=-1)
    return out.reshape(N, 1, OUT_FEATS)
